# baseline (speedup 1.0000x reference)
import functools

import jax
import jax.numpy as jnp
from jax import lax
from jax.experimental import pallas as pl
from jax.experimental.pallas import tpu as pltpu
from jax.experimental.pallas import tpu_sc as plsc

N_COLS = 256
N_ROWS = 65536
N_SEL = 64

NUM_CORES = 2
NUM_SUBCORES = 16
NW = NUM_CORES * NUM_SUBCORES
CHUNK = N_ROWS // NW
BATCH = 8
NBATCH = N_SEL // BATCH
NBUF = 5
NSPM = 2


def _sc_gather(values2, cols):
  mesh = plsc.VectorSubcoreMesh(core_axis_name="c", subcore_axis_name="s")

  @functools.partial(
      pl.kernel,
      mesh=mesh,
      out_type=jax.ShapeDtypeStruct((N_SEL, N_ROWS), jnp.float32),
      scratch_types=[
          pltpu.VMEM((N_SEL,), jnp.int32),
          pltpu.VMEM((NBUF, BATCH, CHUNK), jnp.float32),
          pltpu.VMEM_SHARED((NUM_SUBCORES, NSPM, BATCH, CHUNK), jnp.float32),
          pltpu.SemaphoreType.DMA,
          pltpu.SemaphoreType.DMA,
          pltpu.SemaphoreType.DMA,
          pltpu.SemaphoreType.DMA,
      ],
  )
  def k(values_hbm, cols_hbm, out_hbm, idx_v, buf_v, spm_v, gsem, ssem,
        c1sem, c2sem):
    sid = lax.axis_index("s")
    w = sid * NUM_CORES + lax.axis_index("c")
    col0 = w * CHUNK

    pltpu.sync_copy(cols_hbm, idx_v)

    def gather(b):
      c = pltpu.make_async_copy(
          values_hbm.at[idx_v.at[pl.ds(b * BATCH, BATCH)], pl.ds(col0, CHUNK)],
          buf_v.at[b % NBUF],
          gsem,
      )
      c.start()
      return c

    def scatter(b):
      s = pltpu.make_async_copy(
          buf_v.at[b % NBUF],
          out_hbm.at[pl.ds(b * BATCH, BATCH), pl.ds(col0, CHUNK)],
          ssem,
      )
      s.start()
      return s

    def to_spmem(b, j):
      c = pltpu.make_async_copy(buf_v.at[b % NBUF], spm_v.at[sid, j], c1sem)
      c.start()
      return c

    def spmem_out(b, j):
      c = pltpu.make_async_copy(
          spm_v.at[sid, j],
          out_hbm.at[pl.ds(b * BATCH, BATCH), pl.ds(col0, CHUNK)],
          c2sem,
      )
      c.start()
      return c

    ndirect = NBATCH - NSPM
    gathers = [gather(b) for b in range(NBUF)]
    scatters = []
    hop1 = []
    for b in range(NBATCH):
      gathers[b].wait()
      if b < ndirect:
        scatters.append(scatter(b))
      else:
        hop1.append(to_spmem(b, b - ndirect))
      if b + NBUF < NBATCH:
        scatters[b].wait()
        gathers.append(gather(b + NBUF))
    hop2 = []
    for j in range(NSPM):
      hop1[j].wait()
      hop2.append(spmem_out(ndirect + j, j))
    for b in range(NBATCH - NBUF, ndirect):
      scatters[b].wait()
    for j in range(NSPM):
      hop2[j].wait()

  return k(values2, cols)


def kernel(values, cols):
  return _sc_gather(values, cols)


# speedup vs baseline: 1.0326x; 1.0326x over previous
import functools

import jax
import jax.numpy as jnp
from jax import lax
from jax.experimental import pallas as pl
from jax.experimental.pallas import tpu as pltpu
from jax.experimental.pallas import tpu_sc as plsc

N_COLS = 256
N_ROWS = 65536
N_SEL = 64

NUM_CORES = 2
NUM_SUBCORES = 16
NW = NUM_CORES * NUM_SUBCORES
CHUNK = N_ROWS // NW
BATCH = 8
NBATCH = N_SEL // BATCH
NSLOT = 6
NPAIR = NBATCH // 2


def _sc_gather(values, cols):
  mesh = plsc.VectorSubcoreMesh(core_axis_name="c", subcore_axis_name="s")

  @functools.partial(
      pl.kernel,
      mesh=mesh,
      out_type=jax.ShapeDtypeStruct((N_SEL, N_ROWS), jnp.float32),
      scratch_types=[
          pltpu.VMEM((N_SEL,), jnp.int32),
          pltpu.VMEM((NSLOT * BATCH, CHUNK), jnp.float32),
          pltpu.SemaphoreType.DMA,
          pltpu.SemaphoreType.DMA,
      ],
  )
  def k(values_hbm, cols_hbm, out_hbm, idx_v, buf_v, gsem, ssem):
    w = lax.axis_index("s") * NUM_CORES + lax.axis_index("c")
    col0 = w * CHUNK

    pltpu.sync_copy(cols_hbm, idx_v)

    def gather(b):
      c = pltpu.make_async_copy(
          values_hbm.at[idx_v.at[pl.ds(b * BATCH, BATCH)], pl.ds(col0, CHUNK)],
          buf_v.at[pl.ds((b % NSLOT) * BATCH, BATCH)],
          gsem,
      )
      c.start()
      return c

    def scatter_pair(p):
      s = pltpu.make_async_copy(
          buf_v.at[pl.ds(((2 * p) % NSLOT) * BATCH, 2 * BATCH)],
          out_hbm.at[pl.ds(p * 2 * BATCH, 2 * BATCH), pl.ds(col0, CHUNK)],
          ssem,
      )
      s.start()
      return s

    gathers = [gather(b) for b in range(NSLOT)]
    gathers[0].wait()
    gathers[1].wait()
    s0 = scatter_pair(0)
    gathers[2].wait()
    gathers[3].wait()
    s1 = scatter_pair(1)
    s0.wait()
    gathers.append(gather(6))
    gathers.append(gather(7))
    gathers[4].wait()
    gathers[5].wait()
    s2 = scatter_pair(2)
    gathers[6].wait()
    gathers[7].wait()
    s3 = scatter_pair(3)
    s1.wait()
    s2.wait()
    s3.wait()

  return k(values, cols)


def kernel(values, cols):
  return _sc_gather(values, cols)
